# baseline (speedup 1.0000x reference)
import jax
import jax.numpy as jnp
from jax import lax
from jax.experimental import pallas as pl
from jax.experimental.pallas import tpu as pltpu
from jax.experimental.pallas import tpu_sc as plsc

_DEPTH = 1000
_OUT = 64
_BATCH = 16384
_NC = 2
_NS = 16
_NW = _NC * _NS
_BPW = _BATCH // _NW
_CHUNK = 128
_NCH = _BPW // _CHUNK
_IDX_ROWS = _BATCH // _CHUNK


def _prep_body(wt_ref, b_ref, table_ref):
    table_ref[...] = wt_ref[...] + b_ref[...]


def _gather_body(table_hbm, idx_hbm, out_hbm, idx_v, rows_v, sem):
    wid = lax.axis_index("s") * _NC + lax.axis_index("c")
    pltpu.sync_copy(idx_hbm.at[pl.ds(wid * _NCH, _NCH)], idx_v)
    cps = [
        pltpu.async_copy(table_hbm.at[idx_v.at[j]],
                         rows_v.at[pl.ds(j * _CHUNK, _CHUNK)], sem)
        for j in range(_NCH)
    ]
    for cp in cps:
        cp.wait()
    pltpu.sync_copy(rows_v, out_hbm.at[pl.ds(wid * _BPW, _BPW)])


def kernel(input_batch, W, b):
    idx = jnp.reshape(input_batch.astype(jnp.int32), (_IDX_ROWS, _CHUNK))
    wt = W.T
    b2 = jnp.reshape(b, (1, _OUT))

    table = pl.pallas_call(
        _prep_body,
        out_shape=jax.ShapeDtypeStruct((_DEPTH, _OUT), jnp.float32),
    )(wt, b2)

    mesh = plsc.VectorSubcoreMesh(core_axis_name="c", subcore_axis_name="s",
                                  num_cores=_NC, num_subcores=_NS)
    gather = pl.kernel(
        _gather_body,
        out_type=jax.ShapeDtypeStruct((_BATCH, _OUT), jnp.float32),
        mesh=mesh,
        scratch_types=[
            pltpu.VMEM((_NCH, _CHUNK), jnp.int32),
            pltpu.VMEM((_BPW, _OUT), jnp.float32),
            pltpu.SemaphoreType.DMA,
        ],
        compiler_params=pltpu.CompilerParams(use_tc_tiling_on_sc=False),
    )
    out = gather(table, idx)
    return out[:, None, :]


# speedup vs baseline: 1.0549x; 1.0549x over previous
import jax
import jax.numpy as jnp
from jax import lax
from jax.experimental import pallas as pl
from jax.experimental.pallas import tpu as pltpu
from jax.experimental.pallas import tpu_sc as plsc

_DEPTH = 1000
_OUT = 64
_BATCH = 16384
_NC = 2
_NS = 16
_NW = _NC * _NS
_BPW = _BATCH // _NW
_CHUNK = 128
_NCH = _BPW // _CHUNK
_IDX_ROWS = _BATCH // _CHUNK
_VPR = _OUT // 16


def _body(wt_hbm, idx_hbm, b_hbm, out_hbm, idx_v, rows_v, b_v, gsems, osems):
    wid = lax.axis_index("s") * _NC + lax.axis_index("c")
    pltpu.sync_copy(b_hbm, b_v)
    pltpu.sync_copy(idx_hbm.at[pl.ds(wid * _NCH, _NCH)], idx_v)
    gathers = [
        pltpu.async_copy(wt_hbm.at[idx_v.at[j]],
                         rows_v.at[pl.ds(j * _CHUNK, _CHUNK)], gsems.at[j])
        for j in range(_NCH)
    ]
    bvecs = [b_v[pl.ds(k * 16, 16)] for k in range(_VPR)]
    writes = []
    for j in range(_NCH):
        gathers[j].wait()

        def add_bias(i, _):
            r = j * _CHUNK + i
            for k in range(_VPR):
                rows_v[r, pl.ds(k * 16, 16)] = (
                    rows_v[r, pl.ds(k * 16, 16)] + bvecs[k])
            return 0

        lax.fori_loop(0, _CHUNK, add_bias, 0, unroll=2)
        writes.append(
            pltpu.async_copy(rows_v.at[pl.ds(j * _CHUNK, _CHUNK)],
                             out_hbm.at[pl.ds(wid * _BPW + j * _CHUNK, _CHUNK)],
                             osems.at[j]))
    for cp in writes:
        cp.wait()


def kernel(input_batch, W, b):
    idx = jnp.reshape(input_batch.astype(jnp.int32), (_IDX_ROWS, _CHUNK))
    wt = W.T

    mesh = plsc.VectorSubcoreMesh(core_axis_name="c", subcore_axis_name="s",
                                  num_cores=_NC, num_subcores=_NS)
    run = pl.kernel(
        _body,
        out_type=jax.ShapeDtypeStruct((_BATCH, _OUT), jnp.float32),
        mesh=mesh,
        scratch_types=[
            pltpu.VMEM((_NCH, _CHUNK), jnp.int32),
            pltpu.VMEM((_BPW, _OUT), jnp.float32),
            pltpu.VMEM((_OUT,), jnp.float32),
            pltpu.SemaphoreType.DMA((_NCH,)),
            pltpu.SemaphoreType.DMA((_NCH,)),
        ],
        compiler_params=pltpu.CompilerParams(use_tc_tiling_on_sc=False),
    )
    out = run(wt, idx, b)
    return out[:, None, :]
